# Initial kernel scaffold; baseline (speedup 1.0000x reference)
import jax
import jax.numpy as jnp
from jax.experimental import pallas as pl


def kernel(x, edge_index, W1, b1, W2, b2, W3, b3):
    raise NotImplementedError("write your pallas kernel here")


# speedup vs baseline: 8.3508x; 8.3508x over previous
import functools

import jax
import jax.numpy as jnp
from jax import lax
from jax.experimental import pallas as pl
from jax.experimental.pallas import tpu as pltpu
from jax.experimental.pallas import tpu_sc as plsc

N_ROWS = 10000
R = 10240
PAD_ROW = 10000
NCORES, NSUB, LANES = 2, 16, 16
NW = NCORES * NSUB
K = 128
NCHUNK = 80
E_PAD = NW * NCHUNK * K
ROWS_PER_TILE = R // NSUB
DW = 16
BR = 1024
GRID = R // BR

_MESH = dict(core_axis_name="c", subcore_axis_name="s",
             num_cores=NCORES, num_subcores=NSUB)


def _zero_fill(buf, rows, width):
    zeros16 = jnp.zeros((LANES,), jnp.float32)

    def zrow(i, _):
        for j in range(width // LANES):
            buf[i, pl.ds(j * LANES, LANES)] = zeros16
        return 0

    lax.fori_loop(0, rows, zrow, 0)


def _make_agg(d):
    mesh = plsc.VectorSubcoreMesh(**_MESH)

    @functools.partial(
        pl.kernel,
        out_type=jax.ShapeDtypeStruct((NCORES, R, d), jnp.float32),
        mesh=mesh,
        compiler_params=pltpu.CompilerParams(use_tc_tiling_on_sc=False),
        scratch_types=[
            pltpu.VMEM((NCHUNK, K), jnp.int32),
            pltpu.VMEM((NCHUNK, K), jnp.int32),
            pltpu.VMEM((K, d), jnp.float32),
            pltpu.VMEM_SHARED((R, d), jnp.float32),
        ],
    )
    def agg(table_hbm, src_hbm, dst_hbm, out_hbm, src_v, dst_v, gbuf, acc_sh):
        c = lax.axis_index("c")
        s = lax.axis_index("s")
        wid = s * NCORES + c
        pltpu.sync_copy(src_hbm.at[wid], src_v)
        pltpu.sync_copy(dst_hbm.at[wid], dst_v)
        _zero_fill(gbuf, K, d)
        for j in range(ROWS_PER_TILE // K):
            pltpu.sync_copy(gbuf, acc_sh.at[pl.ds(s * ROWS_PER_TILE + j * K, K)])
        plsc.subcore_barrier()

        def body(g, _):
            pltpu.sync_copy(table_hbm.at[src_v.at[g]], gbuf)
            pltpu.sync_copy(gbuf, acc_sh.at[dst_v.at[g]], add=True)
            return 0

        lax.fori_loop(0, NCHUNK, body, 0)
        plsc.subcore_barrier()
        row0 = s * ROWS_PER_TILE
        pltpu.sync_copy(acc_sh.at[pl.ds(row0, ROWS_PER_TILE)],
                        out_hbm.at[c, pl.ds(row0, ROWS_PER_TILE)])

    return agg


def _make_deg():
    mesh = plsc.VectorSubcoreMesh(**_MESH)

    @functools.partial(
        pl.kernel,
        out_type=jax.ShapeDtypeStruct((NCORES, R, DW), jnp.float32),
        mesh=mesh,
        compiler_params=pltpu.CompilerParams(use_tc_tiling_on_sc=False),
        scratch_types=[
            pltpu.VMEM((NCHUNK, K), jnp.int32),
            pltpu.VMEM((K, DW), jnp.float32),
            pltpu.VMEM_SHARED((R, DW), jnp.float32),
        ],
    )
    def deg(dst_hbm, out_hbm, dst_v, obuf, deg_sh):
        c = lax.axis_index("c")
        s = lax.axis_index("s")
        wid = s * NCORES + c
        pltpu.sync_copy(dst_hbm.at[wid], dst_v)
        _zero_fill(obuf, K, DW)
        for j in range(ROWS_PER_TILE // K):
            pltpu.sync_copy(obuf, deg_sh.at[pl.ds(s * ROWS_PER_TILE + j * K, K)])
        ones16 = jnp.ones((LANES,), jnp.float32)

        def orow(i, _):
            obuf[i, pl.ds(0, LANES)] = ones16
            return 0

        lax.fori_loop(0, K, orow, 0)
        plsc.subcore_barrier()

        def body(g, _):
            pltpu.sync_copy(obuf, deg_sh.at[dst_v.at[g]], add=True)
            return 0

        lax.fori_loop(0, NCHUNK, body, 0)
        plsc.subcore_barrier()
        row0 = s * ROWS_PER_TILE
        pltpu.sync_copy(deg_sh.at[pl.ds(row0, ROWS_PER_TILE)],
                        out_hbm.at[c, pl.ds(row0, ROWS_PER_TILE)])

    return deg


_agg128 = _make_agg(128)
_agg64 = _make_agg(64)
_deg = _make_deg()


def _tc1(deg2, xp, W1):
    def body(deg_r, x_r, w_r, dinv_r, hp_r):
        degv = deg_r[0, :, 0:1] + deg_r[1, :, 0:1] + 1.0
        dinv = lax.rsqrt(degv)
        h = jnp.dot(x_r[...], w_r[...], preferred_element_type=jnp.float32)
        dinv_r[...] = dinv
        hp_r[...] = h * dinv

    return pl.pallas_call(
        body,
        grid=(GRID,),
        in_specs=[
            pl.BlockSpec((NCORES, BR, DW), lambda i: (0, i, 0)),
            pl.BlockSpec((BR, 128), lambda i: (i, 0)),
            pl.BlockSpec((128, 128), lambda i: (0, 0)),
        ],
        out_specs=[
            pl.BlockSpec((BR, 1), lambda i: (i, 0)),
            pl.BlockSpec((BR, 128), lambda i: (i, 0)),
        ],
        out_shape=[
            jax.ShapeDtypeStruct((R, 1), jnp.float32),
            jax.ShapeDtypeStruct((R, 128), jnp.float32),
        ],
    )(deg2, xp, W1)


def _tc_mid(acc, hp, dinv, b, W, dout):
    def body(acc_r, hp_r, dinv_r, b_r, w_r, o_r):
        comb = acc_r[0] + acc_r[1] + hp_r[...]
        z = jnp.maximum(comb * dinv_r[...] + b_r[...], 0.0)
        o_r[...] = jnp.dot(z, w_r[...], preferred_element_type=jnp.float32) * dinv_r[...]

    return pl.pallas_call(
        body,
        grid=(GRID,),
        in_specs=[
            pl.BlockSpec((NCORES, BR, 128), lambda i: (0, i, 0)),
            pl.BlockSpec((BR, 128), lambda i: (i, 0)),
            pl.BlockSpec((BR, 1), lambda i: (i, 0)),
            pl.BlockSpec((1, 128), lambda i: (0, 0)),
            pl.BlockSpec((128, dout), lambda i: (0, 0)),
        ],
        out_specs=pl.BlockSpec((BR, dout), lambda i: (i, 0)),
        out_shape=jax.ShapeDtypeStruct((R, dout), jnp.float32),
    )(acc, hp, dinv, b, W)


def _tc_fin(acc, hp, dinv, b3):
    def body(acc_r, hp_r, dinv_r, b_r, o_r):
        y = (acc_r[0] + acc_r[1] + hp_r[...]) * dinv_r[...] + b_r[...]
        m = jnp.max(y, axis=-1, keepdims=True)
        lse = jnp.log(jnp.sum(jnp.exp(y - m), axis=-1, keepdims=True))
        o_r[...] = y - m - lse

    return pl.pallas_call(
        body,
        grid=(GRID,),
        in_specs=[
            pl.BlockSpec((NCORES, BR, 64), lambda i: (0, i, 0)),
            pl.BlockSpec((BR, 64), lambda i: (i, 0)),
            pl.BlockSpec((BR, 1), lambda i: (i, 0)),
            pl.BlockSpec((1, 64), lambda i: (0, 0)),
        ],
        out_specs=pl.BlockSpec((BR, 64), lambda i: (i, 0)),
        out_shape=jax.ShapeDtypeStruct((R, 64), jnp.float32),
    )(acc, hp, dinv, b3)


@jax.jit
def kernel(x, edge_index, W1, b1, W2, b2, W3, b3):
    ei = edge_index.astype(jnp.int32)
    e = ei.shape[1]
    pad = jnp.full((E_PAD - e,), PAD_ROW, jnp.int32)
    srcp = jnp.concatenate([ei[0], pad]).reshape(NW, NCHUNK, K)
    dstp = jnp.concatenate([ei[1], pad]).reshape(NW, NCHUNK, K)
    xp = jnp.pad(x, ((0, R - x.shape[0]), (0, 0)))

    deg2 = _deg(dstp)
    dinv, hp1 = _tc1(deg2, xp, W1)
    acc1 = _agg128(hp1, srcp, dstp)
    hp2 = _tc_mid(acc1, hp1, dinv, b1.reshape(1, 128), W2, 128)
    acc2 = _agg128(hp2, srcp, dstp)
    hp3 = _tc_mid(acc2, hp2, dinv, b2.reshape(1, 128), W3, 64)
    acc3 = _agg64(hp3, srcp, dstp)
    out = _tc_fin(acc3, hp3, dinv, b3.reshape(1, 64))
    return out[:N_ROWS]


# speedup vs baseline: 9.2313x; 1.1054x over previous
import functools

import jax
import jax.numpy as jnp
from jax import lax
from jax.experimental import pallas as pl
from jax.experimental.pallas import tpu as pltpu
from jax.experimental.pallas import tpu_sc as plsc

N_ROWS = 10000
R = 10240
PAD_ROW = 10000
NCORES, NSUB, LANES = 2, 16, 16
K = 128
NCHUNK = 160
NBUF = 4
NCHUNK_T = NCHUNK + NBUF
ROWS_PER_TILE = R // NSUB
DW = 16
DEG_CHUNK = NCHUNK // NCORES
BR = 1024
GRID = R // BR

_MESH = dict(core_axis_name="c", subcore_axis_name="s",
             num_cores=NCORES, num_subcores=NSUB)


def _zero_fill(buf, rows, width):
    zeros16 = jnp.zeros((LANES,), jnp.float32)

    def zrow(i, _):
        for j in range(width // LANES):
            buf[i, pl.ds(j * LANES, LANES)] = zeros16
        return 0

    lax.fori_loop(0, rows, zrow, 0)


def _make_agg(dh):
    mesh = plsc.VectorSubcoreMesh(**_MESH)

    @functools.partial(
        pl.kernel,
        out_type=jax.ShapeDtypeStruct((NCORES, R, dh), jnp.float32),
        mesh=mesh,
        compiler_params=pltpu.CompilerParams(use_tc_tiling_on_sc=False),
        scratch_types=[
            pltpu.VMEM((NCHUNK_T, K), jnp.int32),
            pltpu.VMEM((NCHUNK_T, K), jnp.int32),
            pltpu.VMEM((NBUF, K, dh), jnp.float32),
            pltpu.VMEM_SHARED((R, dh), jnp.float32),
            pltpu.SemaphoreType.DMA((NBUF,)),
        ],
    )
    def agg(table_hbm, src_hbm, dst_hbm, out_hbm, src_v, dst_v, gbuf, acc_sh, gsem):
        c = lax.axis_index("c")
        s = lax.axis_index("s")
        pltpu.sync_copy(src_hbm.at[s], src_v)
        pltpu.sync_copy(dst_hbm.at[s], dst_v)
        off = (c * R).astype(jnp.int32)

        def arow(i, _):
            for j in range(K // LANES):
                sl = pl.ds(j * LANES, LANES)
                src_v[i, sl] = src_v[i, sl] + off
            return 0

        lax.fori_loop(0, NCHUNK_T, arow, 0)
        _zero_fill(gbuf.at[0], K, dh)
        for j in range(ROWS_PER_TILE // K):
            pltpu.sync_copy(gbuf.at[0], acc_sh.at[pl.ds(s * ROWS_PER_TILE + j * K, K)])
        plsc.subcore_barrier()

        def fire(b, i):
            pltpu.async_copy(table_hbm.at[src_v.at[i]], gbuf.at[b], gsem.at[b])

        def drain(b, i):
            pltpu.make_async_copy(table_hbm.at[src_v.at[i]], gbuf.at[b],
                                  gsem.at[b]).wait()

        for b in range(NBUF):
            fire(b, b)

        def body(outer, _):
            base = outer * NBUF
            for b in range(NBUF):
                i = base + b
                drain(b, i)
                pltpu.sync_copy(gbuf.at[b], acc_sh.at[dst_v.at[i]], add=True)
                fire(b, i + NBUF)
            return 0

        lax.fori_loop(0, NCHUNK // NBUF, body, 0)
        for b in range(NBUF):
            drain(b, b)
        plsc.subcore_barrier()
        row0 = s * ROWS_PER_TILE
        pltpu.sync_copy(acc_sh.at[pl.ds(row0, ROWS_PER_TILE)],
                        out_hbm.at[c, pl.ds(row0, ROWS_PER_TILE)])

    return agg


def _make_deg():
    mesh = plsc.VectorSubcoreMesh(**_MESH)

    @functools.partial(
        pl.kernel,
        out_type=jax.ShapeDtypeStruct((NCORES, R, DW), jnp.float32),
        mesh=mesh,
        compiler_params=pltpu.CompilerParams(use_tc_tiling_on_sc=False),
        scratch_types=[
            pltpu.VMEM((DEG_CHUNK, K), jnp.int32),
            pltpu.VMEM((K, DW), jnp.float32),
            pltpu.VMEM_SHARED((R, DW), jnp.float32),
        ],
    )
    def deg(dst_hbm, out_hbm, dst_v, obuf, deg_sh):
        c = lax.axis_index("c")
        s = lax.axis_index("s")
        pltpu.sync_copy(dst_hbm.at[s, pl.ds(c * DEG_CHUNK, DEG_CHUNK)], dst_v)
        _zero_fill(obuf, K, DW)
        for j in range(ROWS_PER_TILE // K):
            pltpu.sync_copy(obuf, deg_sh.at[pl.ds(s * ROWS_PER_TILE + j * K, K)])
        ones16 = jnp.ones((LANES,), jnp.float32)

        def orow(i, _):
            obuf[i, pl.ds(0, LANES)] = ones16
            return 0

        lax.fori_loop(0, K, orow, 0)
        plsc.subcore_barrier()

        def body(g, _):
            pltpu.sync_copy(obuf, deg_sh.at[dst_v.at[g]], add=True)
            return 0

        lax.fori_loop(0, DEG_CHUNK, body, 0)
        plsc.subcore_barrier()
        row0 = s * ROWS_PER_TILE
        pltpu.sync_copy(deg_sh.at[pl.ds(row0, ROWS_PER_TILE)],
                        out_hbm.at[c, pl.ds(row0, ROWS_PER_TILE)])

    return deg


_agg64 = _make_agg(64)
_agg32 = _make_agg(32)
_deg = _make_deg()


def _tc1(deg2, xp, W1):
    def body(deg_r, x_r, w_r, dinv_r, hp_r):
        degv = deg_r[0, :, 0:1] + deg_r[1, :, 0:1] + 1.0
        dinv = lax.rsqrt(degv)
        h = jnp.dot(x_r[...], w_r[...], preferred_element_type=jnp.float32) * dinv
        dinv_r[...] = dinv
        hp_r[0] = h[:, :64]
        hp_r[1] = h[:, 64:]

    return pl.pallas_call(
        body,
        grid=(GRID,),
        in_specs=[
            pl.BlockSpec((NCORES, BR, DW), lambda i: (0, i, 0)),
            pl.BlockSpec((BR, 128), lambda i: (i, 0)),
            pl.BlockSpec((128, 128), lambda i: (0, 0)),
        ],
        out_specs=[
            pl.BlockSpec((BR, 1), lambda i: (i, 0)),
            pl.BlockSpec((NCORES, BR, 64), lambda i: (0, i, 0)),
        ],
        out_shape=[
            jax.ShapeDtypeStruct((R, 1), jnp.float32),
            jax.ShapeDtypeStruct((NCORES, R, 64), jnp.float32),
        ],
    )(deg2, xp, W1)


def _tc_mid(acc, hp, dinv, b, W, dout):
    def body(acc_r, hp_r, dinv_r, b_r, w_r, o_r):
        comb = jnp.concatenate([acc_r[0] + hp_r[0], acc_r[1] + hp_r[1]], axis=-1)
        z = jnp.maximum(comb * dinv_r[...] + b_r[...], 0.0)
        h = jnp.dot(z, w_r[...], preferred_element_type=jnp.float32) * dinv_r[...]
        o_r[0] = h[:, :dout // 2]
        o_r[1] = h[:, dout // 2:]

    return pl.pallas_call(
        body,
        grid=(GRID,),
        in_specs=[
            pl.BlockSpec((NCORES, BR, 64), lambda i: (0, i, 0)),
            pl.BlockSpec((NCORES, BR, 64), lambda i: (0, i, 0)),
            pl.BlockSpec((BR, 1), lambda i: (i, 0)),
            pl.BlockSpec((1, 128), lambda i: (0, 0)),
            pl.BlockSpec((128, dout), lambda i: (0, 0)),
        ],
        out_specs=pl.BlockSpec((NCORES, BR, dout // 2), lambda i: (0, i, 0)),
        out_shape=jax.ShapeDtypeStruct((NCORES, R, dout // 2), jnp.float32),
    )(acc, hp, dinv, b, W)


def _tc_fin(acc, hp, dinv, b3):
    def body(acc_r, hp_r, dinv_r, b_r, o_r):
        comb = jnp.concatenate([acc_r[0] + hp_r[0], acc_r[1] + hp_r[1]], axis=-1)
        y = comb * dinv_r[...] + b_r[...]
        m = jnp.max(y, axis=-1, keepdims=True)
        lse = jnp.log(jnp.sum(jnp.exp(y - m), axis=-1, keepdims=True))
        o_r[...] = y - m - lse

    return pl.pallas_call(
        body,
        grid=(GRID,),
        in_specs=[
            pl.BlockSpec((NCORES, BR, 32), lambda i: (0, i, 0)),
            pl.BlockSpec((NCORES, BR, 32), lambda i: (0, i, 0)),
            pl.BlockSpec((BR, 1), lambda i: (i, 0)),
            pl.BlockSpec((1, 64), lambda i: (0, 0)),
        ],
        out_specs=pl.BlockSpec((BR, 64), lambda i: (i, 0)),
        out_shape=jax.ShapeDtypeStruct((R, 64), jnp.float32),
    )(acc, hp, dinv, b3)


@jax.jit
def kernel(x, edge_index, W1, b1, W2, b2, W3, b3):
    ei = edge_index.astype(jnp.int32)
    e = ei.shape[1]
    pad = jnp.full((NSUB * NCHUNK * K - e,), PAD_ROW, jnp.int32)
    tail = jnp.full((NSUB, NBUF, K), PAD_ROW, jnp.int32)
    srcp = jnp.concatenate(
        [jnp.concatenate([ei[0], pad]).reshape(NSUB, NCHUNK, K), tail], axis=1)
    dstp = jnp.concatenate(
        [jnp.concatenate([ei[1], pad]).reshape(NSUB, NCHUNK, K), tail], axis=1)
    xp = jnp.pad(x, ((0, R - x.shape[0]), (0, 0)))

    deg2 = _deg(dstp)
    dinv, hp1 = _tc1(deg2, xp, W1)
    acc1 = _agg64(hp1.reshape(NCORES * R, 64), srcp, dstp)
    hp2 = _tc_mid(acc1, hp1, dinv, b1.reshape(1, 128), W2, 128)
    acc2 = _agg64(hp2.reshape(NCORES * R, 64), srcp, dstp)
    hp3 = _tc_mid(acc2, hp2, dinv, b2.reshape(1, 128), W3, 64)
    acc3 = _agg32(hp3.reshape(NCORES * R, 32), srcp, dstp)
    out = _tc_fin(acc3, hp3, dinv, b3.reshape(1, 64))
    return out[:N_ROWS]


# speedup vs baseline: 21.9127x; 2.3737x over previous
import functools

import jax
import jax.numpy as jnp
from jax import lax
from jax.experimental import pallas as pl
from jax.experimental.pallas import tpu as pltpu
from jax.experimental.pallas import tpu_sc as plsc

N_ROWS = 10000
R = 10240
PAD_ROW = 10000
NCORES, NSUB, LANES = 2, 16, 16
K = 128
NCHUNK = 160
HALF = NCHUNK // 2
NBUF = 2
ROWS_PER_TILE = R // NSUB
DW = 16
DEG_CHUNK = NCHUNK // NCORES
BR = 1024
GRID = R // BR

_MESH = dict(core_axis_name="c", subcore_axis_name="s",
             num_cores=NCORES, num_subcores=NSUB)


def _zero_fill(buf, rows, width):
    zeros16 = jnp.zeros((LANES,), jnp.float32)

    def zrow(i, _):
        for j in range(width // LANES):
            buf[i, pl.ds(j * LANES, LANES)] = zeros16
        return 0

    lax.fori_loop(0, rows, zrow, 0)


def _make_agg(dh):
    mesh = plsc.VectorSubcoreMesh(**_MESH)

    @functools.partial(
        pl.kernel,
        out_type=jax.ShapeDtypeStruct((NCORES, R, dh), jnp.float32),
        mesh=mesh,
        compiler_params=pltpu.CompilerParams(use_tc_tiling_on_sc=False),
        scratch_types=[
            pltpu.VMEM((HALF, K), jnp.int32),
            pltpu.VMEM((HALF, K), jnp.int32),
            pltpu.VMEM((NBUF, K, dh), jnp.float32),
            pltpu.VMEM_SHARED((R, dh), jnp.float32),
            pltpu.VMEM_SHARED((R, dh), jnp.float32),
            pltpu.SemaphoreType.DMA((NBUF,)),
        ],
    )
    def agg(table_hbm, src_hbm, dst_hbm, out_hbm, src_v, dst_v, gbuf,
            tab_sh, acc_sh, gsem):
        c = lax.axis_index("c")
        s = lax.axis_index("s")
        row0 = s * ROWS_PER_TILE
        pltpu.sync_copy(table_hbm.at[c, pl.ds(row0, ROWS_PER_TILE)],
                        tab_sh.at[pl.ds(row0, ROWS_PER_TILE)])
        _zero_fill(gbuf.at[0], K, dh)
        for j in range(ROWS_PER_TILE // K):
            pltpu.sync_copy(gbuf.at[0], acc_sh.at[pl.ds(row0 + j * K, K)])
        plsc.subcore_barrier()

        def fire(b, i):
            pltpu.async_copy(tab_sh.at[src_v.at[i]], gbuf.at[b], gsem.at[b])

        def drain(b, i):
            pltpu.make_async_copy(tab_sh.at[src_v.at[i]], gbuf.at[b],
                                  gsem.at[b]).wait()

        def scat(b, i):
            pltpu.sync_copy(gbuf.at[b], acc_sh.at[dst_v.at[i]], add=True)

        for p in range(2):
            pltpu.sync_copy(src_hbm.at[s, pl.ds(p * HALF, HALF)], src_v)
            pltpu.sync_copy(dst_hbm.at[s, pl.ds(p * HALF, HALF)], dst_v)
            for b in range(NBUF):
                fire(b, b)

            def body(outer, _):
                base = outer * NBUF
                for b in range(NBUF):
                    i = base + b
                    drain(b, i)
                    scat(b, i)
                    fire(b, i + NBUF)
                return 0

            lax.fori_loop(0, HALF // NBUF - 1, body, 0)
            for b in range(NBUF):
                i = HALF - NBUF + b
                drain(b, i)
                scat(b, i)

        plsc.subcore_barrier()
        pltpu.sync_copy(acc_sh.at[pl.ds(row0, ROWS_PER_TILE)],
                        out_hbm.at[c, pl.ds(row0, ROWS_PER_TILE)])

    return agg


def _make_deg():
    mesh = plsc.VectorSubcoreMesh(**_MESH)

    @functools.partial(
        pl.kernel,
        out_type=jax.ShapeDtypeStruct((NCORES, R, DW), jnp.float32),
        mesh=mesh,
        compiler_params=pltpu.CompilerParams(use_tc_tiling_on_sc=False),
        scratch_types=[
            pltpu.VMEM((DEG_CHUNK, K), jnp.int32),
            pltpu.VMEM((K, DW), jnp.float32),
            pltpu.VMEM_SHARED((R, DW), jnp.float32),
        ],
    )
    def deg(dst_hbm, out_hbm, dst_v, obuf, deg_sh):
        c = lax.axis_index("c")
        s = lax.axis_index("s")
        pltpu.sync_copy(dst_hbm.at[s, pl.ds(c * DEG_CHUNK, DEG_CHUNK)], dst_v)
        _zero_fill(obuf, K, DW)
        for j in range(ROWS_PER_TILE // K):
            pltpu.sync_copy(obuf, deg_sh.at[pl.ds(s * ROWS_PER_TILE + j * K, K)])
        ones16 = jnp.ones((LANES,), jnp.float32)

        def orow(i, _):
            obuf[i, pl.ds(0, LANES)] = ones16
            return 0

        lax.fori_loop(0, K, orow, 0)
        plsc.subcore_barrier()

        def body(g, _):
            pltpu.sync_copy(obuf, deg_sh.at[dst_v.at[g]], add=True)
            return 0

        lax.fori_loop(0, DEG_CHUNK, body, 0)
        plsc.subcore_barrier()
        row0 = s * ROWS_PER_TILE
        pltpu.sync_copy(deg_sh.at[pl.ds(row0, ROWS_PER_TILE)],
                        out_hbm.at[c, pl.ds(row0, ROWS_PER_TILE)])

    return deg


_agg64 = _make_agg(64)
_agg32 = _make_agg(32)
_deg = _make_deg()


def _tc1(deg2, xp, W1):
    def body(deg_r, x_r, w_r, dinv_r, hp_r):
        degv = deg_r[0, :, 0:1] + deg_r[1, :, 0:1] + 1.0
        dinv = lax.rsqrt(degv)
        h = jnp.dot(x_r[...], w_r[...], preferred_element_type=jnp.float32) * dinv
        dinv_r[...] = dinv
        hp_r[0] = h[:, :64]
        hp_r[1] = h[:, 64:]

    return pl.pallas_call(
        body,
        grid=(GRID,),
        in_specs=[
            pl.BlockSpec((NCORES, BR, DW), lambda i: (0, i, 0)),
            pl.BlockSpec((BR, 128), lambda i: (i, 0)),
            pl.BlockSpec((128, 128), lambda i: (0, 0)),
        ],
        out_specs=[
            pl.BlockSpec((BR, 1), lambda i: (i, 0)),
            pl.BlockSpec((NCORES, BR, 64), lambda i: (0, i, 0)),
        ],
        out_shape=[
            jax.ShapeDtypeStruct((R, 1), jnp.float32),
            jax.ShapeDtypeStruct((NCORES, R, 64), jnp.float32),
        ],
    )(deg2, xp, W1)


def _tc_mid(acc, hp, dinv, b, W, dout):
    def body(acc_r, hp_r, dinv_r, b_r, w_r, o_r):
        comb = jnp.concatenate([acc_r[0] + hp_r[0], acc_r[1] + hp_r[1]], axis=-1)
        z = jnp.maximum(comb * dinv_r[...] + b_r[...], 0.0)
        h = jnp.dot(z, w_r[...], preferred_element_type=jnp.float32) * dinv_r[...]
        o_r[0] = h[:, :dout // 2]
        o_r[1] = h[:, dout // 2:]

    return pl.pallas_call(
        body,
        grid=(GRID,),
        in_specs=[
            pl.BlockSpec((NCORES, BR, 64), lambda i: (0, i, 0)),
            pl.BlockSpec((NCORES, BR, 64), lambda i: (0, i, 0)),
            pl.BlockSpec((BR, 1), lambda i: (i, 0)),
            pl.BlockSpec((1, 128), lambda i: (0, 0)),
            pl.BlockSpec((128, dout), lambda i: (0, 0)),
        ],
        out_specs=pl.BlockSpec((NCORES, BR, dout // 2), lambda i: (0, i, 0)),
        out_shape=jax.ShapeDtypeStruct((NCORES, R, dout // 2), jnp.float32),
    )(acc, hp, dinv, b, W)


def _tc_fin(acc, hp, dinv, b3):
    def body(acc_r, hp_r, dinv_r, b_r, o_r):
        comb = jnp.concatenate([acc_r[0] + hp_r[0], acc_r[1] + hp_r[1]], axis=-1)
        y = comb * dinv_r[...] + b_r[...]
        m = jnp.max(y, axis=-1, keepdims=True)
        lse = jnp.log(jnp.sum(jnp.exp(y - m), axis=-1, keepdims=True))
        o_r[...] = y - m - lse

    return pl.pallas_call(
        body,
        grid=(GRID,),
        in_specs=[
            pl.BlockSpec((NCORES, BR, 32), lambda i: (0, i, 0)),
            pl.BlockSpec((NCORES, BR, 32), lambda i: (0, i, 0)),
            pl.BlockSpec((BR, 1), lambda i: (i, 0)),
            pl.BlockSpec((1, 64), lambda i: (0, 0)),
        ],
        out_specs=pl.BlockSpec((BR, 64), lambda i: (i, 0)),
        out_shape=jax.ShapeDtypeStruct((R, 64), jnp.float32),
    )(acc, hp, dinv, b3)


@jax.jit
def kernel(x, edge_index, W1, b1, W2, b2, W3, b3):
    ei = edge_index.astype(jnp.int32)
    e = ei.shape[1]
    pad = jnp.full((NSUB * NCHUNK * K - e,), PAD_ROW, jnp.int32)
    srcp = jnp.concatenate([ei[0], pad]).reshape(NSUB, NCHUNK, K)
    dstp = jnp.concatenate([ei[1], pad]).reshape(NSUB, NCHUNK, K)
    xp = jnp.pad(x, ((0, R - x.shape[0]), (0, 0)))

    deg2 = _deg(dstp)
    dinv, hp1 = _tc1(deg2, xp, W1)
    acc1 = _agg64(hp1, srcp, dstp)
    hp2 = _tc_mid(acc1, hp1, dinv, b1.reshape(1, 128), W2, 128)
    acc2 = _agg64(hp2, srcp, dstp)
    hp3 = _tc_mid(acc2, hp2, dinv, b2.reshape(1, 128), W3, 64)
    acc3 = _agg32(hp3, srcp, dstp)
    out = _tc_fin(acc3, hp3, dinv, b3.reshape(1, 64))
    return out[:N_ROWS]


# speedup vs baseline: 24.8504x; 1.1341x over previous
import functools

import jax
import jax.numpy as jnp
from jax import lax
from jax.experimental import pallas as pl
from jax.experimental.pallas import tpu as pltpu
from jax.experimental.pallas import tpu_sc as plsc

N_ROWS = 10000
R = 10240
PAD_ROW = 10000
NCORES, NSUB, LANES = 2, 16, 16
K = 128
NCHUNK = 160
HALF = NCHUNK // 2
NBUF = 2
ROWS_PER_TILE = R // NSUB
DW = 16
DEG_CHUNK = NCHUNK // NCORES
BR = 1024
GRID = R // BR

_MESH = dict(core_axis_name="c", subcore_axis_name="s",
             num_cores=NCORES, num_subcores=NSUB)


def _zero_fill(buf, rows, width):
    zeros16 = jnp.zeros((LANES,), jnp.float32)

    def zrow(i, _):
        for j in range(width // LANES):
            buf[i, pl.ds(j * LANES, LANES)] = zeros16
        return 0

    lax.fori_loop(0, rows, zrow, 0)


def _make_agg(dh):
    mesh = plsc.VectorSubcoreMesh(**_MESH)

    @functools.partial(
        pl.kernel,
        out_type=jax.ShapeDtypeStruct((R, 128), jnp.float32),
        mesh=mesh,
        compiler_params=pltpu.CompilerParams(use_tc_tiling_on_sc=False),
        scratch_types=[
            pltpu.VMEM((HALF, K), jnp.int32),
            pltpu.VMEM((HALF, K), jnp.int32),
            pltpu.VMEM((NBUF, K, dh), jnp.float32),
            pltpu.VMEM_SHARED((R, dh), jnp.float32),
            pltpu.VMEM_SHARED((R, dh), jnp.float32),
            pltpu.SemaphoreType.DMA((NBUF,)),
        ],
    )
    def agg(table_hbm, src_hbm, dst_hbm, out_hbm, src_v, dst_v, gbuf,
            tab_sh, acc_sh, gsem):
        c = lax.axis_index("c")
        s = lax.axis_index("s")
        row0 = s * ROWS_PER_TILE
        col0 = c * dh
        pltpu.sync_copy(table_hbm.at[pl.ds(row0, ROWS_PER_TILE), pl.ds(col0, dh)],
                        tab_sh.at[pl.ds(row0, ROWS_PER_TILE)])
        _zero_fill(gbuf.at[0], K, dh)
        for j in range(ROWS_PER_TILE // K):
            pltpu.sync_copy(gbuf.at[0], acc_sh.at[pl.ds(row0 + j * K, K)])
        plsc.subcore_barrier()

        def fire(b, i):
            pltpu.async_copy(tab_sh.at[src_v.at[i]], gbuf.at[b], gsem.at[b])

        def drain(b, i):
            pltpu.make_async_copy(tab_sh.at[src_v.at[i]], gbuf.at[b],
                                  gsem.at[b]).wait()

        def scat(b, i):
            pltpu.sync_copy(gbuf.at[b], acc_sh.at[dst_v.at[i]], add=True)

        for p in range(2):
            pltpu.sync_copy(src_hbm.at[s, pl.ds(p * HALF, HALF)], src_v)
            pltpu.sync_copy(dst_hbm.at[s, pl.ds(p * HALF, HALF)], dst_v)
            for b in range(NBUF):
                fire(b, b)

            def body(outer, _):
                base = outer * NBUF
                for b in range(NBUF):
                    i = base + b
                    drain(b, i)
                    scat(b, i)
                    fire(b, i + NBUF)
                return 0

            lax.fori_loop(0, HALF // NBUF - 1, body, 0)
            for b in range(NBUF):
                i = HALF - NBUF + b
                drain(b, i)
                scat(b, i)

        plsc.subcore_barrier()
        pltpu.sync_copy(acc_sh.at[pl.ds(row0, ROWS_PER_TILE)],
                        out_hbm.at[pl.ds(row0, ROWS_PER_TILE), pl.ds(col0, dh)])

    return agg


def _make_deg():
    mesh = plsc.VectorSubcoreMesh(**_MESH)

    @functools.partial(
        pl.kernel,
        out_type=jax.ShapeDtypeStruct((R, 128), jnp.float32),
        mesh=mesh,
        compiler_params=pltpu.CompilerParams(use_tc_tiling_on_sc=False),
        scratch_types=[
            pltpu.VMEM((DEG_CHUNK, K), jnp.int32),
            pltpu.VMEM((K, DW), jnp.float32),
            pltpu.VMEM_SHARED((R, DW), jnp.float32),
        ],
    )
    def deg(dst_hbm, out_hbm, dst_v, obuf, deg_sh):
        c = lax.axis_index("c")
        s = lax.axis_index("s")
        row0 = s * ROWS_PER_TILE
        pltpu.sync_copy(dst_hbm.at[s, pl.ds(c * DEG_CHUNK, DEG_CHUNK)], dst_v)
        _zero_fill(obuf, K, DW)
        for j in range(ROWS_PER_TILE // K):
            pltpu.sync_copy(obuf, deg_sh.at[pl.ds(row0 + j * K, K)])
        ones16 = jnp.ones((LANES,), jnp.float32)

        def orow(i, _):
            obuf[i, pl.ds(0, LANES)] = ones16
            return 0

        lax.fori_loop(0, K, orow, 0)
        plsc.subcore_barrier()

        def body(g, _):
            pltpu.sync_copy(obuf, deg_sh.at[dst_v.at[g]], add=True)
            return 0

        lax.fori_loop(0, DEG_CHUNK, body, 0)
        plsc.subcore_barrier()
        pltpu.sync_copy(deg_sh.at[pl.ds(row0, ROWS_PER_TILE)],
                        out_hbm.at[pl.ds(row0, ROWS_PER_TILE), pl.ds(c * DW, DW)])

    return deg


_agg64 = _make_agg(64)
_agg32 = _make_agg(32)
_deg = _make_deg()


def _tc1(deg2, xp, W1):
    def body(deg_r, x_r, w_r, dinv_r, hp_r):
        degv = deg_r[:, 0:1] + deg_r[:, 16:17] + 1.0
        dinv = lax.rsqrt(degv)
        h = jnp.dot(x_r[...], w_r[...], preferred_element_type=jnp.float32) * dinv
        dinv_r[...] = dinv
        hp_r[...] = h

    return pl.pallas_call(
        body,
        grid=(GRID,),
        in_specs=[
            pl.BlockSpec((BR, 128), lambda i: (i, 0)),
            pl.BlockSpec((BR, 128), lambda i: (i, 0)),
            pl.BlockSpec((128, 128), lambda i: (0, 0)),
        ],
        out_specs=[
            pl.BlockSpec((BR, 1), lambda i: (i, 0)),
            pl.BlockSpec((BR, 128), lambda i: (i, 0)),
        ],
        out_shape=[
            jax.ShapeDtypeStruct((R, 1), jnp.float32),
            jax.ShapeDtypeStruct((R, 128), jnp.float32),
        ],
    )(deg2, xp, W1)


def _tc_mid(acc, hp, dinv, b, W, dout):
    def body(acc_r, hp_r, dinv_r, b_r, w_r, o_r):
        comb = acc_r[...] + hp_r[...]
        z = jnp.maximum(comb * dinv_r[...] + b_r[...], 0.0)
        h = jnp.dot(z, w_r[...], preferred_element_type=jnp.float32) * dinv_r[...]
        if dout < 128:
            h = jnp.concatenate(
                [h, jnp.zeros((BR, 128 - dout), jnp.float32)], axis=-1)
        o_r[...] = h

    return pl.pallas_call(
        body,
        grid=(GRID,),
        in_specs=[
            pl.BlockSpec((BR, 128), lambda i: (i, 0)),
            pl.BlockSpec((BR, 128), lambda i: (i, 0)),
            pl.BlockSpec((BR, 1), lambda i: (i, 0)),
            pl.BlockSpec((1, 128), lambda i: (0, 0)),
            pl.BlockSpec((128, dout), lambda i: (0, 0)),
        ],
        out_specs=pl.BlockSpec((BR, 128), lambda i: (i, 0)),
        out_shape=jax.ShapeDtypeStruct((R, 128), jnp.float32),
    )(acc, hp, dinv, b, W)


def _tc_fin(acc, hp, dinv, b3):
    def body(acc_r, hp_r, dinv_r, b_r, o_r):
        y = (acc_r[:, :64] + hp_r[:, :64]) * dinv_r[...] + b_r[...]
        m = jnp.max(y, axis=-1, keepdims=True)
        lse = jnp.log(jnp.sum(jnp.exp(y - m), axis=-1, keepdims=True))
        o_r[...] = y - m - lse

    return pl.pallas_call(
        body,
        grid=(GRID,),
        in_specs=[
            pl.BlockSpec((BR, 128), lambda i: (i, 0)),
            pl.BlockSpec((BR, 128), lambda i: (i, 0)),
            pl.BlockSpec((BR, 1), lambda i: (i, 0)),
            pl.BlockSpec((1, 64), lambda i: (0, 0)),
        ],
        out_specs=pl.BlockSpec((BR, 64), lambda i: (i, 0)),
        out_shape=jax.ShapeDtypeStruct((R, 64), jnp.float32),
    )(acc, hp, dinv, b3)


@jax.jit
def kernel(x, edge_index, W1, b1, W2, b2, W3, b3):
    ei = edge_index.astype(jnp.int32)
    e = ei.shape[1]
    pad = jnp.full((NSUB * NCHUNK * K - e,), PAD_ROW, jnp.int32)
    srcp = jnp.concatenate([ei[0], pad]).reshape(NSUB, NCHUNK, K)
    dstp = jnp.concatenate([ei[1], pad]).reshape(NSUB, NCHUNK, K)
    xp = jnp.pad(x, ((0, R - x.shape[0]), (0, 0)))

    deg2 = _deg(dstp)
    dinv, hp1 = _tc1(deg2, xp, W1)
    acc1 = _agg64(hp1, srcp, dstp)
    hp2 = _tc_mid(acc1, hp1, dinv, b1.reshape(1, 128), W2, 128)
    acc2 = _agg64(hp2, srcp, dstp)
    hp3 = _tc_mid(acc2, hp2, dinv, b2.reshape(1, 128), W3, 64)
    acc3 = _agg32(hp3, srcp, dstp)
    out = _tc_fin(acc3, hp3, dinv, b3.reshape(1, 64))
    return out[:N_ROWS]
